# Initial kernel scaffold; baseline (speedup 1.0000x reference)
import jax
import jax.numpy as jnp
from jax.experimental import pallas as pl


def kernel(x, token_table, pos_table):
    raise NotImplementedError("write your pallas kernel here")


# speedup vs baseline: 1.4916x; 1.4916x over previous
import functools

import jax
import jax.numpy as jnp
from jax import lax
from jax.experimental import pallas as pl
from jax.experimental.pallas import tpu as pltpu
from jax.experimental.pallas import tpu_sc as plsc

_B = 4096
_L = 200
_D = 32
_FLAT = _B * _L
_NW = 32
_PER_W = _FLAT // _NW
_C = 1600
_CHUNKS = _PER_W // _C
_PERIODS = _C // _L

_SUBS = [(s, min(128, _C - s)) for s in range(0, _C, 128)]

_mesh = plsc.VectorSubcoreMesh(core_axis_name="c", subcore_axis_name="s")


@functools.partial(
    pl.kernel,
    mesh=_mesh,
    compiler_params=pltpu.CompilerParams(use_tc_tiling_on_sc=False),
    out_type=jax.ShapeDtypeStruct((_FLAT, _D), jnp.float32),
    scratch_types=[
        pltpu.VMEM((_C,), jnp.int32),
        pltpu.VMEM((_C,), jnp.int32),
        pltpu.VMEM((_C, _D), jnp.float32),
        pltpu.VMEM((_C, _D), jnp.float32),
        pltpu.VMEM((_L, _D), jnp.float32),
        pltpu.SemaphoreType.DMA,
        pltpu.SemaphoreType.DMA,
        pltpu.SemaphoreType.DMA,
        pltpu.SemaphoreType.DMA,
        pltpu.SemaphoreType.DMA,
        pltpu.SemaphoreType.DMA,
    ],
)
def _sc_embed(x_ref, tok_ref, pos_ref, out_ref,
              idx0, idx1, rows0, rows1, pos_v,
              si0, si1, sg0, sg1, so0, so1):
    wid = lax.axis_index("s") * 2 + lax.axis_index("c")
    base = wid * _PER_W

    idxb = (idx0, idx1)
    rowsb = (rows0, rows1)
    si = (si0, si1)
    sg = (sg0, sg1)
    so = (so0, so1)

    pltpu.sync_copy(pos_ref, pos_v)

    def start_idx(g, b):
        return pltpu.async_copy(
            x_ref.at[pl.ds(base + g * _C, _C)], idxb[b], si[b])

    def start_gathers(b):
        return [
            pltpu.async_copy(
                tok_ref.at[idxb[b].at[pl.ds(s, n)]],
                rowsb[b].at[pl.ds(s, n)],
                sg[b])
            for (s, n) in _SUBS
        ]

    def start_scat(g, b):
        return pltpu.async_copy(
            rowsb[b], out_ref.at[pl.ds(base + g * _C, _C)], so[b])

    def do_add(b):
        rb = rowsb[b]

        def lbody(l, carry):
            p0 = pos_v[l, pl.ds(0, 16)]
            p1 = pos_v[l, pl.ds(16, 16)]
            for k in range(_PERIODS):
                r = l + _L * k
                rb[r, pl.ds(0, 16)] += p0
                rb[r, pl.ds(16, 16)] += p1
            return carry

        lax.fori_loop(0, _L, lbody, 0)

    cpi = [start_idx(0, 0), start_idx(1, 1)]
    cpi[0].wait()
    cpg = [start_gathers(0), None]
    cpo = [None, None]
    for g in range(_CHUNKS):
        b = g & 1
        nb = 1 - b
        for c in cpg[b]:
            c.wait()
        if g + 2 < _CHUNKS:
            cpi[b] = start_idx(g + 2, b)
        if g + 1 < _CHUNKS:
            cpi[nb].wait()
            if cpo[nb] is not None:
                cpo[nb].wait()
            cpg[nb] = start_gathers(nb)
        do_add(b)
        cpo[b] = start_scat(g, b)
    cpo[0].wait()
    cpo[1].wait()


def kernel(x, token_table, pos_table):
    xf = x.reshape(-1).astype(jnp.int32)
    out = _sc_embed(xf, token_table, pos_table)
    return out.reshape(_B, _L, _D)


# speedup vs baseline: 1.4936x; 1.0013x over previous
import functools

import jax
import jax.numpy as jnp
from jax import lax
from jax.experimental import pallas as pl
from jax.experimental.pallas import tpu as pltpu
from jax.experimental.pallas import tpu_sc as plsc

_B = 4096
_L = 200
_D = 32
_FLAT = _B * _L
_NW = 32
_PER_W = _FLAT // _NW
_C = 1600
_CHUNKS = _PER_W // _C
_PERIODS = _C // _L

_SUBS = [(0, _C)]

_mesh = plsc.VectorSubcoreMesh(core_axis_name="c", subcore_axis_name="s")


@functools.partial(
    pl.kernel,
    mesh=_mesh,
    compiler_params=pltpu.CompilerParams(use_tc_tiling_on_sc=False),
    out_type=jax.ShapeDtypeStruct((_FLAT, _D), jnp.float32),
    scratch_types=[
        pltpu.VMEM((_C,), jnp.int32),
        pltpu.VMEM((_C,), jnp.int32),
        pltpu.VMEM((_C, _D), jnp.float32),
        pltpu.VMEM((_C, _D), jnp.float32),
        pltpu.VMEM((_L, _D), jnp.float32),
        pltpu.SemaphoreType.DMA,
        pltpu.SemaphoreType.DMA,
        pltpu.SemaphoreType.DMA,
        pltpu.SemaphoreType.DMA,
        pltpu.SemaphoreType.DMA,
        pltpu.SemaphoreType.DMA,
    ],
)
def _sc_embed(x_ref, tok_ref, pos_ref, out_ref,
              idx0, idx1, rows0, rows1, pos_v,
              si0, si1, sg0, sg1, so0, so1):
    wid = lax.axis_index("s") * 2 + lax.axis_index("c")
    base = wid * _PER_W

    idxb = (idx0, idx1)
    rowsb = (rows0, rows1)
    si = (si0, si1)
    sg = (sg0, sg1)
    so = (so0, so1)

    pltpu.sync_copy(pos_ref, pos_v)

    def start_idx(g, b):
        return pltpu.async_copy(
            x_ref.at[pl.ds(base + g * _C, _C)], idxb[b], si[b])

    def start_gathers(b):
        return [
            pltpu.async_copy(
                tok_ref.at[idxb[b].at[pl.ds(s, n)]],
                rowsb[b].at[pl.ds(s, n)],
                sg[b])
            for (s, n) in _SUBS
        ]

    def start_scat(g, b):
        return pltpu.async_copy(
            rowsb[b], out_ref.at[pl.ds(base + g * _C, _C)], so[b])

    def do_add(b):
        rb = rowsb[b]

        def lbody(l, carry):
            p0 = pos_v[l, pl.ds(0, 16)]
            p1 = pos_v[l, pl.ds(16, 16)]
            for k in range(_PERIODS):
                r = l + _L * k
                rb[r, pl.ds(0, 16)] += p0
                rb[r, pl.ds(16, 16)] += p1
            return carry

        lax.fori_loop(0, _L, lbody, 0)

    cpi = [start_idx(0, 0), start_idx(1, 1)]
    cpi[0].wait()
    cpg = [start_gathers(0), None]
    cpo = [None, None]
    for g in range(_CHUNKS):
        b = g & 1
        nb = 1 - b
        for c in cpg[b]:
            c.wait()
        if g + 2 < _CHUNKS:
            cpi[b] = start_idx(g + 2, b)
        if g + 1 < _CHUNKS:
            cpi[nb].wait()
            if cpo[nb] is not None:
                cpo[nb].wait()
            cpg[nb] = start_gathers(nb)
        do_add(b)
        cpo[b] = start_scat(g, b)
    cpo[0].wait()
    cpo[1].wait()


def kernel(x, token_table, pos_table):
    xf = x.reshape(-1).astype(jnp.int32)
    out = _sc_embed(xf, token_table, pos_table)
    return out.reshape(_B, _L, _D)
